# baseline (speedup 1.0000x reference)
import jax
import jax.numpy as jnp
from jax.experimental import pallas as pl

_BATCH = 64
_SEQ_LEN = 262144
_MAX_DROP_LENGTH = 2048
_BLK = 4096


def _drop_bounds(batch, seq_len):
    rkey = jax.random.key(42)
    k_start, k_len = jax.random.split(rkey)
    drop_start = jax.random.randint(k_start, (batch,), 0, seq_len // 2)
    drop_len = jax.random.randint(k_len, (batch,), 0, _MAX_DROP_LENGTH)
    drop_end = jnp.minimum(drop_start + drop_len, seq_len)
    return drop_start.astype(jnp.int32), drop_end.astype(jnp.int32)


def _masked_copy_kernel(start_ref, end_ref, x_ref, o_ref):
    j = pl.program_id(0)
    s = start_ref[:, 0:1]
    e = end_ref[:, 0:1]
    col = jax.lax.broadcasted_iota(jnp.int32, (_BATCH, _BLK), 1) + j * _BLK
    mask = (col >= s) & (col < e)
    o_ref[...] = jnp.where(mask, jnp.zeros((), x_ref.dtype), x_ref[...])


def kernel(waveform):
    batch, seq_len = waveform.shape
    drop_start, drop_end = _drop_bounds(batch, seq_len)
    starts2d = jnp.broadcast_to(drop_start[:, None], (batch, 128))
    ends2d = jnp.broadcast_to(drop_end[:, None], (batch, 128))
    grid = (seq_len // _BLK,)
    return pl.pallas_call(
        _masked_copy_kernel,
        grid=grid,
        in_specs=[
            pl.BlockSpec((batch, 128), lambda j: (0, 0)),
            pl.BlockSpec((batch, 128), lambda j: (0, 0)),
            pl.BlockSpec((batch, _BLK), lambda j: (0, j)),
        ],
        out_specs=pl.BlockSpec((batch, _BLK), lambda j: (0, j)),
        out_shape=jax.ShapeDtypeStruct((batch, seq_len), waveform.dtype),
    )(starts2d, ends2d, waveform)
